# baseline (speedup 1.0000x reference)
import functools
import math

import jax
import jax.numpy as jnp
from jax import lax
from jax.experimental import pallas as pl

BN = 256
HEADS = 4
RATIO = 0.5
LVL = 3


def _ceil_to(x, m):
    return ((x + m - 1) // m) * m


def _acc_rows(n):
    return _ceil_to(n + 1, 256)


def _pad_rows(x, n_acc):
    return jnp.pad(x, ((0, n_acc - x.shape[0]), (0, 0)))


def _build_m(ei, ew, n, na):
    e = ei.shape[1]
    valid = ew > 0
    spare = na - n
    dummy = n + (jnp.arange(e, dtype=jnp.int32) % spare)
    r = jnp.where(valid, ei[0].astype(jnp.int32), dummy)
    c = jnp.where(valid, ei[1].astype(jnp.int32), dummy)
    slab = max(256, (14_000_000 // (4 * na)) // 256 * 256)
    slab = min(slab, na)
    nslab = na // slab
    ones = jnp.ones((e,), jnp.float32)
    pieces = []
    for s in range(nslab):
        base = s * slab
        inslab = (c >= base) & (c < base + slab)
        flat_s = jnp.where(inslab, (c - base) * na + r, slab * na)
        piece = jnp.zeros(((slab + 1) * na,), jnp.float32).at[flat_s].add(ones)
        pieces.append(piece.reshape(slab + 1, na)[:slab])
    return jnp.concatenate(pieces, axis=0)


def _deg_dense(m, n):
    na = m.shape[0]

    def body(m_ref, o_ref):
        col = lax.broadcasted_iota(jnp.int32, (BN, na), 1)
        mm = jnp.where(col < n, m_ref[...], 0.0)
        o_ref[...] = jnp.sum(mm, axis=1)[None, :]

    return pl.pallas_call(
        body,
        grid=(na // BN,),
        in_specs=[pl.BlockSpec((BN, na), lambda i: (i, 0))],
        out_specs=pl.BlockSpec((1, BN), lambda i: (0, i)),
        out_shape=jax.ShapeDtypeStruct((1, na), jnp.float32),
    )(m)


def _mm_scale(x_pad, w, deg1, fill):
    na = x_pad.shape[0]

    def body(x_ref, w_ref, d_ref, o_ref):
        dinv = lax.rsqrt(d_ref[0] + fill)
        h = jnp.dot(x_ref[...], w_ref[...], preferred_element_type=jnp.float32)
        o_ref[...] = h * dinv[:, None]

    return pl.pallas_call(
        body,
        grid=(na // BN,),
        in_specs=[pl.BlockSpec((BN, 128), lambda i: (i, 0)),
                  pl.BlockSpec((128, 128), lambda i: (0, 0)),
                  pl.BlockSpec((1, BN), lambda i: (0, i))],
        out_specs=pl.BlockSpec((BN, 128), lambda i: (i, 0)),
        out_shape=jax.ShapeDtypeStruct((na, 128), jnp.float32),
    )(x_pad, w, deg1)


def _prop_mm(m, hp, deg1, b, fill, relu):
    na = hp.shape[0]

    def body(m_ref, h_ref, hb_ref, d_ref, b_ref, o_ref):
        dinv = lax.rsqrt(d_ref[0] + fill)
        acc = jnp.dot(m_ref[...], h_ref[...],
                      preferred_element_type=jnp.float32)
        o = (acc + fill * hb_ref[...]) * dinv[:, None] + b_ref[...]
        if relu:
            o = jnp.maximum(o, 0.0)
        o_ref[...] = o

    return pl.pallas_call(
        body,
        grid=(na // BN,),
        in_specs=[pl.BlockSpec((BN, na), lambda i: (i, 0)),
                  pl.BlockSpec((na, 128), lambda i: (0, 0)),
                  pl.BlockSpec((BN, 128), lambda i: (i, 0)),
                  pl.BlockSpec((1, BN), lambda i: (0, i)),
                  pl.BlockSpec((1, 128), lambda i: (0, 0))],
        out_specs=pl.BlockSpec((BN, 128), lambda i: (i, 0)),
        out_shape=jax.ShapeDtypeStruct((na, 128), jnp.float32),
    )(m, hp, hp, deg1, b.reshape(1, 128))


def _gcn(x_pad, m, deg1, p, fill, relu):
    hp = _mm_scale(x_pad, p['W'], deg1, fill)
    return _prop_mm(m, hp, deg1, p['b'], fill, relu)


def _attn_tail(Qp, Kd, Vd, p):
    dv = Qp.shape[-1]
    split = lambda t: jnp.concatenate(jnp.split(t, HEADS, axis=2), axis=0)
    Q_, K_, V_ = split(Qp), split(Kd), split(Vd)
    A = jax.nn.softmax(jnp.matmul(Q_, jnp.swapaxes(K_, 1, 2)) / math.sqrt(dv),
                       axis=-1)
    out = Q_ + jnp.matmul(A, V_)
    out = jnp.concatenate(jnp.split(out, HEADS, axis=0), axis=2)
    return out + jax.nn.relu(out @ p['o']['W'] + p['o']['b'])


def _mab_dense(Q, K, p):
    Qp = Q @ p['q']['W'] + p['q']['b']
    Kd = K @ p['k']['W'] + p['k']['b']
    Vd = K @ p['v']['W'] + p['v']['b']
    return _attn_tail(Qp, Kd, Vd, p)


def kernel(x, edge_index, edge_weight, params):
    n0 = x.shape[0]
    ew = jnp.ones((edge_index.shape[1],), x.dtype)
    na0 = _acc_rows(n0)
    m0 = _build_m(edge_index, ew, n0, na0)
    deg0 = _deg_dense(m0, n0)

    xp = _pad_rows(x, na0)
    xp = _gcn(xp, m0, deg0, params['down'][0], 2.0, True)

    xs = [xp]
    ns = [n0]
    ms = [m0]
    degs = [deg0]
    perms = []

    cur_ei, cur_ew, n_cur = edge_index, ew, n0
    for i in range(1, LVL + 1):
        xf = xp[:n_cur]
        w = params['pool'][i - 1]
        score = jnp.tanh((xf @ w) / jnp.linalg.norm(w))
        k = int(math.ceil(RATIO * n_cur))
        vals, perm = lax.top_k(score, k)
        x_new = xf[perm] * vals[:, None]
        node_idx = jnp.full((n_cur,), -1, jnp.int32).at[perm].set(
            jnp.arange(k, dtype=jnp.int32))
        nr = node_idx[cur_ei[0]]
        ncol = node_idx[cur_ei[1]]
        valid = (nr >= 0) & (ncol >= 0)
        cur_ei = jnp.stack([jnp.where(valid, nr, 0),
                            jnp.where(valid, ncol, 0)]).astype(cur_ei.dtype)
        cur_ew = jnp.where(valid, cur_ew, 0.0)
        n_cur = k

        na = _acc_rows(k)
        m = _build_m(cur_ei, cur_ew, k, na)
        deg = _deg_dense(m, k)
        xp = _pad_rows(x_new, na)
        xp = _gcn(xp, m, deg, params['down'][i], 2.0, True)
        if i < LVL:
            xs.append(xp)
            ns.append(k)
            ms.append(m)
            degs.append(deg)
        perms.append(perm)

    for i in range(LVL):
        j = LVL - 1 - i
        kj = perms[j].shape[0]
        xt = xp[:kj]
        up = jnp.zeros((ns[j], 128), jnp.float32).at[perms[j]].set(xt)
        xsum = xs[j][:ns[j]] + up
        xp = _pad_rows(xsum, _acc_rows(ns[j]))
        xp = _gcn(xp, ms[j], degs[j], params['up'][i], 2.0, i < LVL - 1)

    g = params['gmt']
    xt = xp[:n0]
    h = xt @ g['lin1']['W'] + g['lin1']['b']
    hp_pad = _pad_rows(h, na0)
    Kd = _gcn(hp_pad, m0, deg0, g['mab_g']['k'], 1.0, False)[:n0][None]
    Vd = _gcn(hp_pad, m0, deg0, g['mab_g']['v'], 1.0, False)[:n0][None]
    Qp = g['S_g'] @ g['mab_g']['q']['W'] + g['mab_g']['q']['b']
    bx = _attn_tail(Qp, Kd, Vd, g['mab_g'])
    bx = _mab_dense(bx, bx, g['mab_s'])
    bx = _mab_dense(g['S_i'], bx, g['mab_i'])
    out = bx[:, 0, :] @ g['lin2']['W'] + g['lin2']['b']
    return out @ params['final']['W'] + params['final']['b']


# speedup vs baseline: 8.6463x; 8.6463x over previous
import functools
import math

import jax
import jax.numpy as jnp
from jax import lax
from jax.experimental import pallas as pl

BN = 256
HEADS = 4
RATIO = 0.5
LVL = 3


def _ceil_to(x, m):
    return ((x + m - 1) // m) * m


def _acc_rows(n):
    return _ceil_to(n + 1, 256)


def _pad_rows(x, n_acc):
    return jnp.pad(x, ((0, n_acc - x.shape[0]), (0, 0)))


def _build_m(ei, ew, n, na):
    e = ei.shape[1]
    valid = ew > 0
    spare = na - n
    dummy = n + (jnp.arange(e, dtype=jnp.int32) % spare)
    r = jnp.where(valid, ei[0].astype(jnp.int32), dummy)
    c = jnp.where(valid, ei[1].astype(jnp.int32), dummy)
    flat = jnp.sort(c * na + r)
    dnums = lax.ScatterDimensionNumbers(
        update_window_dims=(),
        inserted_window_dims=(0,),
        scatter_dims_to_operand_dims=(0,),
    )
    m = lax.scatter_add(
        jnp.zeros((na * na,), jnp.float32), flat[:, None],
        jnp.ones((e,), jnp.float32), dnums,
        indices_are_sorted=True, unique_indices=False)
    return m.reshape(na, na)


def _pool_m(m_prev, perm, n_prev, na):
    k = perm.shape[0]
    perm_pad = jnp.concatenate(
        [perm.astype(jnp.int32),
         jnp.full((na - k,), n_prev, jnp.int32)])
    return jnp.take(jnp.take(m_prev, perm_pad, axis=0), perm_pad, axis=1)


def _deg_dense(m, n):
    na = m.shape[0]

    def body(m_ref, o_ref):
        col = lax.broadcasted_iota(jnp.int32, (BN, na), 1)
        mm = jnp.where(col < n, m_ref[...], 0.0)
        o_ref[...] = jnp.sum(mm, axis=1)[None, :]

    return pl.pallas_call(
        body,
        grid=(na // BN,),
        in_specs=[pl.BlockSpec((BN, na), lambda i: (i, 0))],
        out_specs=pl.BlockSpec((1, BN), lambda i: (0, i)),
        out_shape=jax.ShapeDtypeStruct((1, na), jnp.float32),
    )(m)


def _mm_scale(x_pad, w, deg1, fill):
    na = x_pad.shape[0]

    def body(x_ref, w_ref, d_ref, o_ref):
        dinv = lax.rsqrt(d_ref[0] + fill)
        h = jnp.dot(x_ref[...], w_ref[...], preferred_element_type=jnp.float32)
        o_ref[...] = h * dinv[:, None]

    return pl.pallas_call(
        body,
        grid=(na // BN,),
        in_specs=[pl.BlockSpec((BN, 128), lambda i: (i, 0)),
                  pl.BlockSpec((128, 128), lambda i: (0, 0)),
                  pl.BlockSpec((1, BN), lambda i: (0, i))],
        out_specs=pl.BlockSpec((BN, 128), lambda i: (i, 0)),
        out_shape=jax.ShapeDtypeStruct((na, 128), jnp.float32),
    )(x_pad, w, deg1)


def _prop_mm(m, hp, deg1, b, fill, relu):
    na = hp.shape[0]

    def body(m_ref, h_ref, hb_ref, d_ref, b_ref, o_ref):
        dinv = lax.rsqrt(d_ref[0] + fill)
        acc = jnp.dot(m_ref[...], h_ref[...],
                      preferred_element_type=jnp.float32)
        o = (acc + fill * hb_ref[...]) * dinv[:, None] + b_ref[...]
        if relu:
            o = jnp.maximum(o, 0.0)
        o_ref[...] = o

    return pl.pallas_call(
        body,
        grid=(na // BN,),
        in_specs=[pl.BlockSpec((BN, na), lambda i: (i, 0)),
                  pl.BlockSpec((na, 128), lambda i: (0, 0)),
                  pl.BlockSpec((BN, 128), lambda i: (i, 0)),
                  pl.BlockSpec((1, BN), lambda i: (0, i)),
                  pl.BlockSpec((1, 128), lambda i: (0, 0))],
        out_specs=pl.BlockSpec((BN, 128), lambda i: (i, 0)),
        out_shape=jax.ShapeDtypeStruct((na, 128), jnp.float32),
    )(m, hp, hp, deg1, b.reshape(1, 128))


def _gcn(x_pad, m, deg1, p, fill, relu):
    hp = _mm_scale(x_pad, p['W'], deg1, fill)
    return _prop_mm(m, hp, deg1, p['b'], fill, relu)


def _attn_tail(Qp, Kd, Vd, p):
    dv = Qp.shape[-1]
    split = lambda t: jnp.concatenate(jnp.split(t, HEADS, axis=2), axis=0)
    Q_, K_, V_ = split(Qp), split(Kd), split(Vd)
    A = jax.nn.softmax(jnp.matmul(Q_, jnp.swapaxes(K_, 1, 2)) / math.sqrt(dv),
                       axis=-1)
    out = Q_ + jnp.matmul(A, V_)
    out = jnp.concatenate(jnp.split(out, HEADS, axis=0), axis=2)
    return out + jax.nn.relu(out @ p['o']['W'] + p['o']['b'])


def _mab_dense(Q, K, p):
    Qp = Q @ p['q']['W'] + p['q']['b']
    Kd = K @ p['k']['W'] + p['k']['b']
    Vd = K @ p['v']['W'] + p['v']['b']
    return _attn_tail(Qp, Kd, Vd, p)


def kernel(x, edge_index, edge_weight, params):
    n0 = x.shape[0]
    ew = jnp.ones((edge_index.shape[1],), x.dtype)
    na0 = _acc_rows(n0)
    m0 = _build_m(edge_index, ew, n0, na0)
    deg0 = _deg_dense(m0, n0)

    xp = _pad_rows(x, na0)
    xp = _gcn(xp, m0, deg0, params['down'][0], 2.0, True)

    xs = [xp]
    ns = [n0]
    ms = [m0]
    degs = [deg0]
    perms = []

    m_prev, n_cur = m0, n0
    for i in range(1, LVL + 1):
        xf = xp[:n_cur]
        w = params['pool'][i - 1]
        score = jnp.tanh((xf @ w) / jnp.linalg.norm(w))
        k = int(math.ceil(RATIO * n_cur))
        vals, perm = lax.top_k(score, k)
        x_new = xf[perm] * vals[:, None]

        na = _acc_rows(k)
        m = _pool_m(m_prev, perm, n_cur, na)
        deg = _deg_dense(m, k)
        m_prev, n_cur = m, k
        xp = _pad_rows(x_new, na)
        xp = _gcn(xp, m, deg, params['down'][i], 2.0, True)
        if i < LVL:
            xs.append(xp)
            ns.append(k)
            ms.append(m)
            degs.append(deg)
        perms.append(perm)

    for i in range(LVL):
        j = LVL - 1 - i
        kj = perms[j].shape[0]
        xt = xp[:kj]
        up = jnp.zeros((ns[j], 128), jnp.float32).at[perms[j]].set(xt)
        xsum = xs[j][:ns[j]] + up
        xp = _pad_rows(xsum, _acc_rows(ns[j]))
        xp = _gcn(xp, ms[j], degs[j], params['up'][i], 2.0, i < LVL - 1)

    g = params['gmt']
    xt = xp[:n0]
    h = xt @ g['lin1']['W'] + g['lin1']['b']
    hp_pad = _pad_rows(h, na0)
    Kd = _gcn(hp_pad, m0, deg0, g['mab_g']['k'], 1.0, False)[:n0][None]
    Vd = _gcn(hp_pad, m0, deg0, g['mab_g']['v'], 1.0, False)[:n0][None]
    Qp = g['S_g'] @ g['mab_g']['q']['W'] + g['mab_g']['q']['b']
    bx = _attn_tail(Qp, Kd, Vd, g['mab_g'])
    bx = _mab_dense(bx, bx, g['mab_s'])
    bx = _mab_dense(g['S_i'], bx, g['mab_i'])
    out = bx[:, 0, :] @ g['lin2']['W'] + g['lin2']['b']
    return out @ params['final']['W'] + params['final']['b']
